# baseline (speedup 1.0000x reference)
import functools

import jax
import jax.numpy as jnp
from jax import lax
from jax.experimental import pallas as pl
from jax.experimental.pallas import tpu as pltpu
from jax.experimental.pallas import tpu_sc as plsc

_D = 100
_P = 112
_NC = 2
_NS = 16
_NW = _NC * _NS
_CHUNK = 128
_NBUF = 2
_DCH = _CHUNK * _D


def _repack(src, dst):
    for r in range(_CHUNK):
        sbase = r * _P
        dbase = r * _D
        for k in range(_D // 16):
            dst[pl.ds(dbase + 16 * k, 16)] = src[r, pl.ds(16 * k, 16)]
        dst[pl.ds(dbase + _D - 16, 16)] = src[r, pl.ds(_D - 16, 16)]


def _gather_sc(x3, tpad):
    nchunks = x3.shape[1]
    mesh = plsc.VectorSubcoreMesh(core_axis_name="c", subcore_axis_name="s")

    @functools.partial(
        pl.kernel,
        out_type=jax.ShapeDtypeStruct((_NW, nchunks, _DCH), jnp.float32),
        mesh=mesh,
        scratch_types=(
            [pltpu.VMEM((nchunks, _CHUNK), jnp.int32)]
            + [pltpu.VMEM((_CHUNK, _P), jnp.float32) for _ in range(_NBUF)]
            + [pltpu.VMEM((_DCH,), jnp.float32) for _ in range(_NBUF)]
            + [pltpu.SemaphoreType.DMA for _ in range(2 * _NBUF)]
        ),
        compiler_params=pltpu.CompilerParams(use_tc_tiling_on_sc=False),
    )
    def k(x_hbm, tbl_hbm, out_hbm, idx_v, *rest):
        bufs = rest[:_NBUF]
        dbufs = rest[_NBUF:2 * _NBUF]
        gsem = rest[2 * _NBUF:3 * _NBUF]
        osem = rest[3 * _NBUF:]
        wid = lax.axis_index("s") * _NC + lax.axis_index("c")
        pltpu.sync_copy(x_hbm.at[wid], idx_v)
        for b in range(_NBUF):
            pltpu.async_copy(tbl_hbm.at[idx_v.at[b]], bufs[b], gsem[b])

        @pl.loop(0, nchunks, step=_NBUF)
        def _(g):
            for b in range(_NBUF):
                cur = g + b
                pltpu.make_async_copy(
                    tbl_hbm.at[idx_v.at[cur]], bufs[b], gsem[b]).wait()

                @pl.when(cur >= _NBUF)
                def _():
                    pltpu.make_async_copy(
                        dbufs[b], out_hbm.at[wid, 0], osem[b]).wait()

                _repack(bufs[b], dbufs[b])
                nxt = cur + _NBUF

                @pl.when(nxt < nchunks)
                def _():
                    pltpu.async_copy(
                        tbl_hbm.at[idx_v.at[nxt]], bufs[b], gsem[b])

                pltpu.async_copy(dbufs[b], out_hbm.at[wid, cur], osem[b])

        for b in range(_NBUF):
            pltpu.make_async_copy(
                dbufs[b], out_hbm.at[wid, 0], osem[b]).wait()

    return k(x3, tpad)


def kernel(X, table):
    n, m = X.shape
    total = n * m
    nchunks = total // (_NW * _CHUNK)
    x3 = X.reshape(_NW, nchunks, _CHUNK).astype(jnp.int32)
    tpad = jnp.pad(table.astype(jnp.float32), ((0, 0), (0, _P - _D)))
    out = _gather_sc(x3, tpad)
    return out.reshape(n, m, _D)


# speedup vs baseline: 2.1239x; 2.1239x over previous
import functools

import jax
import jax.numpy as jnp
from jax import lax
from jax.experimental import pallas as pl
from jax.experimental.pallas import tpu as pltpu
from jax.experimental.pallas import tpu_sc as plsc

_D = 100
_P = 128
_NC = 2
_NS = 16
_NW = _NC * _NS
_CHUNK = 128
_NBUF = 5


def _gather_sc(x3, tpad):
    nchunks = x3.shape[1]
    mesh = plsc.VectorSubcoreMesh(core_axis_name="c", subcore_axis_name="s")

    @functools.partial(
        pl.kernel,
        out_type=jax.ShapeDtypeStruct((_NW, nchunks, _CHUNK, _P), jnp.float32),
        mesh=mesh,
        scratch_types=(
            [pltpu.VMEM((nchunks, _CHUNK), jnp.int32)]
            + [pltpu.VMEM((_CHUNK, _P), jnp.float32) for _ in range(_NBUF)]
            + [pltpu.SemaphoreType.DMA for _ in range(2 * _NBUF)]
        ),
        compiler_params=pltpu.CompilerParams(use_tc_tiling_on_sc=False),
    )
    def k(x_hbm, tbl_hbm, out_hbm, idx_v, *rest):
        bufs = rest[:_NBUF]
        gsem = rest[_NBUF:2 * _NBUF]
        osem = rest[2 * _NBUF:]
        wid = lax.axis_index("s") * _NC + lax.axis_index("c")
        pltpu.sync_copy(x_hbm.at[wid], idx_v)
        for b in range(_NBUF):
            pltpu.async_copy(tbl_hbm.at[idx_v.at[b]], bufs[b], gsem[b])

        @pl.loop(0, nchunks, step=_NBUF)
        def _(g):
            for b in range(_NBUF):
                cur = g + b
                pltpu.make_async_copy(
                    tbl_hbm.at[idx_v.at[cur]], bufs[b], gsem[b]).wait()
                pltpu.async_copy(bufs[b], out_hbm.at[wid, cur], osem[b])
                nxt = cur + _NBUF

                @pl.when(nxt < nchunks)
                def _():
                    pltpu.make_async_copy(
                        bufs[b], out_hbm.at[wid, cur], osem[b]).wait()
                    pltpu.async_copy(
                        tbl_hbm.at[idx_v.at[nxt]], bufs[b], gsem[b])

        for b in range(_NBUF):
            pltpu.make_async_copy(
                bufs[b], out_hbm.at[wid, 0], osem[b]).wait()

    return k(x3, tpad)


def kernel(X, table):
    n, m = X.shape
    total = n * m
    nchunks = total // (_NW * _CHUNK)
    x3 = X.reshape(_NW, nchunks, _CHUNK).astype(jnp.int32)
    tpad = jnp.pad(table.astype(jnp.float32), ((0, 0), (0, _P - _D)))
    out = _gather_sc(x3, tpad)
    return out.reshape(n, m, _P)[..., :_D]
